# baseline (speedup 1.0000x reference)
import functools

import jax
import jax.numpy as jnp
from jax import lax
from jax.experimental import pallas as pl
from jax.experimental.pallas import tpu as pltpu
from jax.experimental.pallas import tpu_sc as plsc

VOCAB = 1000000
EMBED_DIM = 64
BATCH = 4096
HIST = 200

NC = 2
NS = 16
NW = NC * NS
B_PER_W = BATCH // NW
PAIR_DIM = 2 * EMBED_DIM
N_CHUNK16 = (HIST + 15) // 16
RAW_PAD = N_CHUNK16 * 16


def _sc_pool_body(x_hbm, table_hbm, out_hbm, raw0, raw1, pair0, pair1,
                  rows_v, pooled_v, isem0, isem1, gsem0, gsem1):
    raws = (raw0, raw1)
    pairs = (pair0, pair1)
    isems = (isem0, isem1)
    gsems = (gsem0, gsem1)
    wid = lax.axis_index("s") * NC + lax.axis_index("c")
    base = wid * B_PER_W

    def issue_idx(row, b):
        pltpu.async_copy(x_hbm.at[pl.ds((base + row) * HIST, HIST)],
                         raws[b].at[pl.ds(0, HIST)], isems[b])

    def wait_idx(b):
        pltpu.make_async_copy(x_hbm.at[pl.ds(0, HIST)],
                              raws[b].at[pl.ds(0, HIST)], isems[b]).wait()

    def partition(b):
        lane = lax.broadcasted_iota(jnp.int32, (16,), 0)
        epos = jnp.zeros((16,), jnp.int32)
        opos = jnp.zeros((16,), jnp.int32)
        for k in range(N_CHUNK16):
            nval = min(HIST - k * 16, 16)
            v = raws[b][pl.ds(k * 16, 16)]
            valid = lane < nval
            in_lo = v < (VOCAB // 2)
            pairv = jnp.where(in_lo, v, v - (VOCAB // 2))
            evm = in_lo & valid
            odm = (~in_lo) & valid
            eidx = plsc.cumsum(evm.astype(jnp.int32))
            ecnt = plsc.all_reduce_population_count(evm)
            ocnt = plsc.all_reduce_population_count(odm)
            vidx = jnp.minimum(lane + 1, nval)
            dest_e = epos + eidx - 1
            dest_o = (HIST - 1) - opos - (vidx - eidx - 1)
            dest = jnp.where(evm, dest_e, dest_o)
            plsc.store_scatter(pairs[b], [dest], pairv, mask=valid)
            epos = epos + ecnt
            opos = opos + ocnt
        return epos

    def issue_gather(b):
        pltpu.async_copy(table_hbm.at[pairs[b]], rows_v.at[b], gsems[b])

    def wait_gather(b):
        pltpu.make_async_copy(table_hbm.at[pl.ds(0, HIST)], rows_v.at[b],
                              gsems[b]).wait()

    issue_idx(0, 0)
    issue_idx(1, 1)
    wait_idx(0)
    ne0 = partition(0)
    issue_gather(0)

    def outer(i, nes):
        for b in range(2):
            row = i * 2 + b
            ne = nes[b]
            wait_gather(b)

            @pl.when(row + 2 < B_PER_W)
            def _():
                issue_idx(row + 2, b)

            def red(l, accs):
                msk = l < ne
                return tuple(
                    accs[j] + jnp.where(
                        msk,
                        rows_v[b, l, pl.ds(j * 16, 16)],
                        rows_v[b, l, pl.ds(EMBED_DIM + j * 16, 16)])
                    for j in range(EMBED_DIM // 16))

            accs = lax.fori_loop(
                0, HIST, red,
                tuple(jnp.zeros((16,), jnp.float32)
                      for _ in range(EMBED_DIM // 16)),
                unroll=8)
            for j in range(EMBED_DIM // 16):
                pooled_v[row, pl.ds(j * 16, 16)] = accs[j] * (1.0 / HIST)

            def prep_next():
                wait_idx(1 - b)
                ne_n = partition(1 - b)
                issue_gather(1 - b)
                return ne_n

            ne_next = lax.cond(row + 1 < B_PER_W, prep_next,
                               lambda: jnp.zeros((16,), jnp.int32))
            nes = (nes[0], ne_next) if b == 0 else (ne_next, nes[1])
        return nes

    lax.fori_loop(0, B_PER_W // 2, outer,
                  (ne0, jnp.zeros((16,), jnp.int32)))
    pltpu.sync_copy(pooled_v, out_hbm.at[pl.ds(base, B_PER_W)])


@jax.jit
def _sc_pool(x, table_c):
    mesh = plsc.VectorSubcoreMesh(core_axis_name="c", subcore_axis_name="s")
    f = functools.partial(
        pl.kernel,
        out_type=jax.ShapeDtypeStruct((BATCH, EMBED_DIM), jnp.float32),
        mesh=mesh,
        compiler_params=pltpu.CompilerParams(use_tc_tiling_on_sc=True,
                                             needs_layout_passes=False),
        scratch_types=[
            pltpu.VMEM((RAW_PAD,), jnp.int32),
            pltpu.VMEM((RAW_PAD,), jnp.int32),
            pltpu.VMEM((HIST,), jnp.int32),
            pltpu.VMEM((HIST,), jnp.int32),
            pltpu.VMEM((2, HIST, PAIR_DIM), jnp.float32),
            pltpu.VMEM((B_PER_W, EMBED_DIM), jnp.float32),
            pltpu.SemaphoreType.DMA,
            pltpu.SemaphoreType.DMA,
            pltpu.SemaphoreType.DMA,
            pltpu.SemaphoreType.DMA,
        ],
    )(_sc_pool_body)
    return f(x, table_c)


DEPAD_BLK = 10000
HALF_V = VOCAB // 2


def _depad_body(t_hbm, o_ref, buf_a, buf_b, sem_a, sem_b):
    i = pl.program_id(0)
    ca = pltpu.make_async_copy(
        t_hbm.at[pl.ds(i * DEPAD_BLK, DEPAD_BLK), :], buf_a, sem_a)
    cb = pltpu.make_async_copy(
        t_hbm.at[pl.ds(HALF_V + i * DEPAD_BLK, DEPAD_BLK), :], buf_b, sem_b)
    ca.start()
    cb.start()
    ca.wait()
    cb.wait()
    o_ref[...] = jnp.concatenate([buf_a[...], buf_b[...]], axis=1)


@jax.jit
def _depad(table):
    nblk = HALF_V // DEPAD_BLK
    return pl.pallas_call(
        _depad_body,
        grid=(nblk,),
        in_specs=[pl.BlockSpec(memory_space=pl.ANY)],
        out_specs=pl.BlockSpec((DEPAD_BLK, PAIR_DIM), lambda i: (i, 0)),
        out_shape=jax.ShapeDtypeStruct((HALF_V, PAIR_DIM), jnp.float32),
        scratch_shapes=[
            pltpu.VMEM((DEPAD_BLK, EMBED_DIM), jnp.float32),
            pltpu.VMEM((DEPAD_BLK, EMBED_DIM), jnp.float32),
            pltpu.SemaphoreType.DMA, pltpu.SemaphoreType.DMA],
    )(table)


def _mlp_body(p_ref, w1_ref, b1_ref, w2_ref, b2_ref, o_ref):
    p = p_ref[...]
    h = lax.dot_general(p, w1_ref[...], (((1,), (1,)), ((), ())),
                        precision=lax.Precision.HIGHEST,
                        preferred_element_type=jnp.float32)
    h = jnp.maximum(h + b1_ref[...], 0.0)
    o_ref[...] = jnp.sum(h * w2_ref[...], axis=1, keepdims=True) + b2_ref[...]


@jax.jit
def _mlp(pooled, W1, b1, W2, b2):
    return pl.pallas_call(
        _mlp_body,
        out_shape=jax.ShapeDtypeStruct((BATCH, 1), jnp.float32),
    )(pooled, W1, b1.reshape(1, 256), W2, b2.reshape(1, 1))


def kernel(x, table, W1, b1, W2, b2):
    table_c = _depad(table)
    x_flat = x.astype(jnp.int32).reshape(BATCH * HIST)
    pooled = _sc_pool(x_flat, table_c)
    return _mlp(pooled, W1, b1, W2, b2)


# speedup vs baseline: 2.0906x; 2.0906x over previous
import functools

import jax
import jax.numpy as jnp
from jax import lax
from jax.experimental import pallas as pl
from jax.experimental.pallas import tpu as pltpu
from jax.experimental.pallas import tpu_sc as plsc
from jax.experimental.layout import Format, Layout, with_layout_constraint

VOCAB = 1000000
EMBED_DIM = 64
BATCH = 4096
HIST = 200

NC = 2
NS = 16
NW = NC * NS
B_PER_W = BATCH // NW


ROWS_PER_CHUNK = 4
CHUNK_IDX = ROWS_PER_CHUNK * HIST
N_CHUNKS = B_PER_W // ROWS_PER_CHUNK


def _sc_pool_body(x_hbm, table_hbm, out_hbm, idx0, idx1, rows_v, pooled_v,
                  isem0, isem1, gsem0, gsem1):
    idxs = (idx0, idx1)
    isems = (isem0, isem1)
    gsems = (gsem0, gsem1)
    wid = lax.axis_index("s") * NC + lax.axis_index("c")
    base = wid * B_PER_W

    def issue_idx(c, b):
        pltpu.async_copy(
            x_hbm.at[pl.ds((base + c * ROWS_PER_CHUNK) * HIST, CHUNK_IDX)],
            idxs[b], isems[b])

    def wait_idx(b):
        pltpu.make_async_copy(x_hbm.at[pl.ds(0, CHUNK_IDX)], idxs[b],
                              isems[b]).wait()

    def issue_gather(b):
        pltpu.async_copy(table_hbm.at[idxs[b]], rows_v.at[b], gsems[b])

    def wait_gather(b):
        pltpu.make_async_copy(table_hbm.at[pl.ds(0, CHUNK_IDX)],
                              rows_v.at[b], gsems[b]).wait()

    issue_idx(0, 0)
    issue_idx(1, 1)
    wait_idx(0)
    issue_gather(0)

    def outer(i, carry):
        for b in range(2):
            c = i * 2 + b
            wait_gather(b)

            @pl.when(c + 2 < N_CHUNKS)
            def _():
                issue_idx(c + 2, b)

            for r in range(ROWS_PER_CHUNK):
                def red(l, accs, _r=r):
                    return tuple(
                        accs[j] + rows_v[b, _r * HIST + l, pl.ds(j * 16, 16)]
                        for j in range(EMBED_DIM // 16))

                accs = lax.fori_loop(
                    0, HIST, red,
                    tuple(jnp.zeros((16,), jnp.float32)
                          for _ in range(EMBED_DIM // 16)),
                    unroll=8)
                row = c * ROWS_PER_CHUNK + r
                for j in range(EMBED_DIM // 16):
                    pooled_v[row, pl.ds(j * 16, 16)] = accs[j] * (1.0 / HIST)

            @pl.when(c + 1 < N_CHUNKS)
            def _():
                wait_idx(1 - b)
                issue_gather(1 - b)
        return carry

    lax.fori_loop(0, N_CHUNKS // 2, outer, 0)
    pltpu.sync_copy(pooled_v, out_hbm.at[pl.ds(base, B_PER_W)])


@jax.jit
def _sc_pool(x_flat, table):
    table = with_layout_constraint(
        table, Layout(major_to_minor=(0, 1), tiling=((8,),)))
    mesh = plsc.VectorSubcoreMesh(core_axis_name="c", subcore_axis_name="s")
    f = functools.partial(
        pl.kernel,
        out_type=jax.ShapeDtypeStruct((BATCH, EMBED_DIM), jnp.float32),
        mesh=mesh,
        compiler_params=pltpu.CompilerParams(use_tc_tiling_on_sc=False),
        scratch_types=[
            pltpu.VMEM((CHUNK_IDX,), jnp.int32),
            pltpu.VMEM((CHUNK_IDX,), jnp.int32),
            pltpu.VMEM((2, CHUNK_IDX, EMBED_DIM), jnp.float32),
            pltpu.VMEM((B_PER_W, EMBED_DIM), jnp.float32),
            pltpu.SemaphoreType.DMA,
            pltpu.SemaphoreType.DMA,
            pltpu.SemaphoreType.DMA,
            pltpu.SemaphoreType.DMA,
        ],
    )(_sc_pool_body)
    return f(x_flat, table)


def _mlp_body(p_ref, w1_ref, b1_ref, w2_ref, b2_ref, o_ref):
    p = p_ref[...]
    h = lax.dot_general(p, w1_ref[...], (((1,), (1,)), ((), ())),
                        precision=lax.Precision.HIGHEST,
                        preferred_element_type=jnp.float32)
    h = jnp.maximum(h + b1_ref[...], 0.0)
    o_ref[...] = jnp.sum(h * w2_ref[...], axis=1, keepdims=True) + b2_ref[...]


@jax.jit
def _mlp(pooled, W1, b1, W2, b2):
    return pl.pallas_call(
        _mlp_body,
        out_shape=jax.ShapeDtypeStruct((BATCH, 1), jnp.float32),
    )(pooled, W1, b1.reshape(1, 256), W2, b2.reshape(1, 1))


def kernel(x, table, W1, b1, W2, b2):
    x_flat = (x.astype(jnp.int32) * 2).reshape(BATCH * HIST)
    pooled = _sc_pool(x_flat, table)
    return _mlp(pooled, W1, b1, W2, b2)
